# baseline (speedup 1.0000x reference)
import functools

import jax
import jax.numpy as jnp
from jax import lax
from jax.experimental import pallas as pl
from jax.experimental.pallas import tpu as pltpu
from jax.experimental.pallas import tpu_sc as plsc

N = 10000
D = 128
E = 320000
NSUB = 16
NCORE = 2
NW = NCORE * NSUB
CHUNK = 128
CH_PER_SUB = 80
E_PAD = NW * CH_PER_SUB * CHUNK
N_PAD = 10112
ROWS_PER_SUB = N_PAD // NSUB
NBUF = 2
PHASES = 2
CH_PHASE = CH_PER_SUB // PHASES

_mesh = plsc.VectorSubcoreMesh(core_axis_name="c", subcore_axis_name="s")


@functools.partial(
    pl.kernel,
    out_type=jax.ShapeDtypeStruct((NCORE, N_PAD, D), jnp.float32),
    mesh=_mesh,
    scratch_types=[
        pltpu.VMEM((CH_PER_SUB, CHUNK), jnp.int32),
        pltpu.VMEM((CHUNK, D), jnp.float32),
        pltpu.VMEM_SHARED((N_PAD, D), jnp.float32),
        pltpu.SemaphoreType.DMA,
    ],
)
def _deg_kernel(col_hbm, ones_hbm, zeros_hbm, out_hbm,
                col_all, ones_v, deg_sh, sem):
    c = lax.axis_index("c")
    s = lax.axis_index("s")
    w = c * NSUB + s
    r0 = s * ROWS_PER_SUB
    pltpu.sync_copy(col_hbm.at[w], col_all)
    pltpu.sync_copy(ones_hbm, ones_v)
    pltpu.sync_copy(zeros_hbm.at[pl.ds(r0, ROWS_PER_SUB), :],
                    deg_sh.at[pl.ds(r0, ROWS_PER_SUB), :])
    plsc.subcore_barrier()

    def block(j, carry):
        descs = [
            pltpu.async_copy(ones_v, deg_sh.at[col_all.at[j * 8 + b]], sem,
                             add=True)
            for b in range(8)
        ]
        for d in descs:
            d.wait()
        return carry

    lax.fori_loop(0, CH_PER_SUB // 8, block, 0)
    plsc.subcore_barrier()
    pltpu.sync_copy(deg_sh.at[pl.ds(r0, ROWS_PER_SUB), :],
                    out_hbm.at[c].at[pl.ds(r0, ROWS_PER_SUB), :])


@functools.partial(
    pl.kernel,
    out_type=jax.ShapeDtypeStruct((NCORE, N_PAD, D), jnp.float32),
    mesh=_mesh,
    scratch_types=[
        pltpu.VMEM((CH_PHASE, CHUNK), jnp.int32),
        pltpu.VMEM((CH_PHASE, CHUNK), jnp.int32),
        pltpu.VMEM((CHUNK, D), jnp.float32),
        pltpu.VMEM((CHUNK, D), jnp.float32),
        pltpu.SemaphoreType.DMA,
        pltpu.SemaphoreType.DMA,
        pltpu.SemaphoreType.DMA,
        pltpu.SemaphoreType.DMA,
        pltpu.VMEM_SHARED((N_PAD, D), jnp.float32),
    ],
)
def _round_kernel(u_hbm, row_hbm, col_hbm, zeros_hbm, acc_hbm,
                  row_all, col_all, buf0, buf1,
                  g0, g1, s0, s1, acc_sh):
    bufs = [buf0, buf1]
    gsems = [g0, g1]
    ssems = [s0, s1]
    c = lax.axis_index("c")
    s = lax.axis_index("s")
    w = c * NSUB + s
    r0 = s * ROWS_PER_SUB

    @pl.when(c == 0)
    def _():
        pltpu.sync_copy(u_hbm.at[pl.ds(r0, ROWS_PER_SUB), :],
                        acc_sh.at[pl.ds(r0, ROWS_PER_SUB), :])

    @pl.when(c == 1)
    def _():
        pltpu.sync_copy(zeros_hbm.at[pl.ds(r0, ROWS_PER_SUB), :],
                        acc_sh.at[pl.ds(r0, ROWS_PER_SUB), :])

    plsc.subcore_barrier()

    for phase in range(PHASES):
        pltpu.sync_copy(row_hbm.at[w].at[pl.ds(phase * CH_PHASE, CH_PHASE)],
                        row_all)
        pltpu.sync_copy(col_hbm.at[w].at[pl.ds(phase * CH_PHASE, CH_PHASE)],
                        col_all)

        def block(j, carry):
            pltpu.async_copy(u_hbm.at[row_all.at[j]], buf0, g0).wait()
            pltpu.sync_copy(buf0, acc_sh.at[col_all.at[j]], add=True)
            return carry

        lax.fori_loop(0, CH_PHASE, block, 0)

    plsc.subcore_barrier()
    pltpu.sync_copy(acc_sh.at[pl.ds(r0, ROWS_PER_SUB), :],
                    acc_hbm.at[c].at[pl.ds(r0, ROWS_PER_SUB), :])


def _prep_body(x_ref, deg_ref, u0_ref, dinv_ref, dinv2_ref):
    deg = deg_ref[0, :, 0:1] + deg_ref[1, :, 0:1] + 1.0
    rows = lax.broadcasted_iota(jnp.int32, (N_PAD, 1), 0)
    dinv = jnp.where(rows < N, lax.rsqrt(deg), 0.0)
    u0_ref[...] = dinv * x_ref[...]
    dinv_ref[...] = dinv
    dinv2_ref[...] = dinv * dinv


def _prep(x_pad, deg2d):
    return pl.pallas_call(
        _prep_body,
        out_shape=(
            jax.ShapeDtypeStruct((N_PAD, D), jnp.float32),
            jax.ShapeDtypeStruct((N_PAD, 1), jnp.float32),
            jax.ShapeDtypeStruct((N_PAD, 1), jnp.float32),
        ),
    )(x_pad, deg2d)


def _scale_body(acc_ref, dinv2_ref, u_ref):
    u_ref[...] = (acc_ref[0] + acc_ref[1]) * dinv2_ref[...]


def _scale(acc, dinv2):
    return pl.pallas_call(
        _scale_body,
        out_shape=jax.ShapeDtypeStruct((N_PAD, D), jnp.float32),
    )(acc, dinv2)


def _final_body(acc_ref, dinv_ref, wt_ref, b_ref, out_ref):
    h = (acc_ref[0] + acc_ref[1]) * dinv_ref[...]
    h = lax.slice(h, (0, 0), (N, D))
    out_ref[...] = jnp.dot(h, wt_ref[...],
                           preferred_element_type=jnp.float32) + b_ref[...]


def _final(acc, dinv, w_t, b2d):
    return pl.pallas_call(
        _final_body,
        out_shape=jax.ShapeDtypeStruct((N, D), jnp.float32),
    )(acc, dinv, w_t, b2d)


def kernel(x, edge_index, W, b):
    row = edge_index[0].astype(jnp.int32)
    col = edge_index[1].astype(jnp.int32)
    pad = E_PAD - row.shape[0]
    row_p = jnp.concatenate([row, jnp.full((pad,), N, jnp.int32)])
    col_p = jnp.concatenate([col, jnp.full((pad,), N, jnp.int32)])
    row3 = row_p.reshape(NW, CH_PER_SUB, CHUNK)
    col3 = col_p.reshape(NW, CH_PER_SUB, CHUNK)
    x_pad = jnp.pad(x, ((0, N_PAD - N), (0, 0)))
    ones_chunk = jnp.ones((CHUNK, D), jnp.float32)
    zeros128 = jnp.zeros((N_PAD, D), jnp.float32)

    deg2d = _deg_kernel(col3, ones_chunk, zeros128)
    u0, dinv, dinv2 = _prep(x_pad, deg2d)
    acc1 = _round_kernel(u0, row3, col3, zeros128)
    u1 = _scale(acc1, dinv2)
    acc2 = _round_kernel(u1, row3, col3, zeros128)
    return _final(acc2, dinv, W.T, b.reshape(1, D))


# speedup vs baseline: 1.1071x; 1.1071x over previous
import functools

import jax
import jax.numpy as jnp
from jax import lax
from jax.experimental import pallas as pl
from jax.experimental.pallas import tpu as pltpu
from jax.experimental.pallas import tpu_sc as plsc

N = 10000
D = 128
E = 320000
NSUB = 16
NCORE = 2
NW = NCORE * NSUB
CHUNK = 128
CH_PER_SUB = 80
E_PAD = NW * CH_PER_SUB * CHUNK
N_PAD = 10112
ROWS_PER_SUB = N_PAD // NSUB
NBUF = 2
PHASES = 2
CH_PHASE = CH_PER_SUB // PHASES

_mesh = plsc.VectorSubcoreMesh(core_axis_name="c", subcore_axis_name="s")


@functools.partial(
    pl.kernel,
    out_type=jax.ShapeDtypeStruct((NCORE, N_PAD, D), jnp.float32),
    mesh=_mesh,
    scratch_types=[
        pltpu.VMEM((CH_PER_SUB, CHUNK), jnp.int32),
        pltpu.VMEM((CHUNK, D), jnp.float32),
        pltpu.VMEM_SHARED((N_PAD, D), jnp.float32),
        pltpu.SemaphoreType.DMA,
    ],
)
def _deg_kernel(col_hbm, ones_hbm, zeros_hbm, out_hbm,
                col_all, ones_v, deg_sh, sem):
    c = lax.axis_index("c")
    s = lax.axis_index("s")
    w = c * NSUB + s
    r0 = s * ROWS_PER_SUB
    pltpu.sync_copy(col_hbm.at[w], col_all)
    pltpu.sync_copy(ones_hbm, ones_v)
    pltpu.sync_copy(zeros_hbm.at[pl.ds(r0, ROWS_PER_SUB), :],
                    deg_sh.at[pl.ds(r0, ROWS_PER_SUB), :])
    plsc.subcore_barrier()

    def block(j, carry):
        descs = [
            pltpu.async_copy(ones_v, deg_sh.at[col_all.at[j * 8 + b]], sem,
                             add=True)
            for b in range(8)
        ]
        for d in descs:
            d.wait()
        return carry

    lax.fori_loop(0, CH_PER_SUB // 8, block, 0)
    plsc.subcore_barrier()
    pltpu.sync_copy(deg_sh.at[pl.ds(r0, ROWS_PER_SUB), :],
                    out_hbm.at[c].at[pl.ds(r0, ROWS_PER_SUB), :])


@functools.partial(
    pl.kernel,
    out_type=jax.ShapeDtypeStruct((NCORE, N_PAD, D), jnp.float32),
    mesh=_mesh,
    scratch_types=[
        pltpu.VMEM((CHUNK,), jnp.int32),
        pltpu.VMEM((CHUNK,), jnp.int32),
        pltpu.VMEM((CHUNK,), jnp.int32),
        pltpu.VMEM((CHUNK,), jnp.int32),
        pltpu.VMEM((CHUNK, D), jnp.float32),
        pltpu.VMEM((CHUNK, D), jnp.float32),
        pltpu.SemaphoreType.DMA,
        pltpu.SemaphoreType.DMA,
        pltpu.SemaphoreType.DMA,
        pltpu.SemaphoreType.DMA,
        pltpu.VMEM_SHARED((N_PAD, D), jnp.float32),
    ],
)
def _round_kernel(u_hbm, row_hbm, col_hbm, zeros_hbm, acc_hbm,
                  rowv0, rowv1, colv0, colv1, buf0, buf1,
                  i0, i1, g0, g1, acc_sh):
    rowv = [rowv0, rowv1]
    colv = [colv0, colv1]
    bufs = [buf0, buf1]
    isems = [i0, i1]
    gsems = [g0, g1]
    c = lax.axis_index("c")
    s = lax.axis_index("s")
    w = c * NSUB + s
    r0 = s * ROWS_PER_SUB

    @pl.when(c == 0)
    def _():
        pltpu.sync_copy(u_hbm.at[pl.ds(r0, ROWS_PER_SUB), :],
                        acc_sh.at[pl.ds(r0, ROWS_PER_SUB), :])

    @pl.when(c == 1)
    def _():
        pltpu.sync_copy(zeros_hbm.at[pl.ds(r0, ROWS_PER_SUB), :],
                        acc_sh.at[pl.ds(r0, ROWS_PER_SUB), :])

    plsc.subcore_barrier()

    e0 = w * CH_PER_SUB * CHUNK
    for b in range(NBUF):
        pltpu.sync_copy(row_hbm.at[pl.ds(e0 + b * CHUNK, CHUNK)], rowv[b])
        pltpu.sync_copy(col_hbm.at[pl.ds(e0 + b * CHUNK, CHUNK)], colv[b])
        pltpu.async_copy(u_hbm.at[rowv[b]], bufs[b], gsems[b])

    def block(j, carry):
        for b in range(NBUF):
            i = j * NBUF + b
            pltpu.make_async_copy(u_hbm.at[rowv[b]], bufs[b], gsems[b]).wait()
            pltpu.sync_copy(bufs[b], acc_sh.at[colv[b]], add=True)
            i_next = i + NBUF

            @pl.when(i_next < CH_PER_SUB)
            def _():
                pltpu.async_copy(
                    row_hbm.at[pl.ds(e0 + i_next * CHUNK, CHUNK)],
                    rowv[b], isems[b])
                pltpu.async_copy(
                    col_hbm.at[pl.ds(e0 + i_next * CHUNK, CHUNK)],
                    colv[b], isems[b])
                pltpu.make_async_copy(
                    row_hbm.at[pl.ds(0, CHUNK)], rowv[b], isems[b]).wait()
                pltpu.make_async_copy(
                    col_hbm.at[pl.ds(0, CHUNK)], colv[b], isems[b]).wait()
                pltpu.async_copy(u_hbm.at[rowv[b]], bufs[b], gsems[b])

        return carry

    lax.fori_loop(0, CH_PER_SUB // NBUF, block, 0)
    plsc.subcore_barrier()
    pltpu.sync_copy(acc_sh.at[pl.ds(r0, ROWS_PER_SUB), :],
                    acc_hbm.at[c].at[pl.ds(r0, ROWS_PER_SUB), :])


def _prep_body(x_ref, deg_ref, u0_ref, dinv_ref, dinv2_ref):
    deg = deg_ref[0, :, 0:1] + deg_ref[1, :, 0:1] + 1.0
    rows = lax.broadcasted_iota(jnp.int32, (N_PAD, 1), 0)
    dinv = jnp.where(rows < N, lax.rsqrt(deg), 0.0)
    u0_ref[...] = dinv * x_ref[...]
    dinv_ref[...] = dinv
    dinv2_ref[...] = dinv * dinv


def _prep(x_pad, deg2d):
    return pl.pallas_call(
        _prep_body,
        out_shape=(
            jax.ShapeDtypeStruct((N_PAD, D), jnp.float32),
            jax.ShapeDtypeStruct((N_PAD, 1), jnp.float32),
            jax.ShapeDtypeStruct((N_PAD, 1), jnp.float32),
        ),
    )(x_pad, deg2d)


def _scale_body(acc_ref, dinv2_ref, u_ref):
    u_ref[...] = (acc_ref[0] + acc_ref[1]) * dinv2_ref[...]


def _scale(acc, dinv2):
    return pl.pallas_call(
        _scale_body,
        out_shape=jax.ShapeDtypeStruct((N_PAD, D), jnp.float32),
    )(acc, dinv2)


def _final_body(acc_ref, dinv_ref, wt_ref, b_ref, out_ref):
    h = (acc_ref[0] + acc_ref[1]) * dinv_ref[...]
    h = lax.slice(h, (0, 0), (N, D))
    out_ref[...] = jnp.dot(h, wt_ref[...],
                           preferred_element_type=jnp.float32) + b_ref[...]


def _final(acc, dinv, w_t, b2d):
    return pl.pallas_call(
        _final_body,
        out_shape=jax.ShapeDtypeStruct((N, D), jnp.float32),
    )(acc, dinv, w_t, b2d)


def kernel(x, edge_index, W, b):
    row = edge_index[0].astype(jnp.int32)
    col = edge_index[1].astype(jnp.int32)
    pad = E_PAD - row.shape[0]
    row_p = jnp.concatenate([row, jnp.full((pad,), N, jnp.int32)])
    col_p = jnp.concatenate([col, jnp.full((pad,), N, jnp.int32)])
    row3 = row_p.reshape(NW, CH_PER_SUB, CHUNK)
    col3 = col_p.reshape(NW, CH_PER_SUB, CHUNK)
    x_pad = jnp.pad(x, ((0, N_PAD - N), (0, 0)))
    ones_chunk = jnp.ones((CHUNK, D), jnp.float32)
    zeros128 = jnp.zeros((N_PAD, D), jnp.float32)

    deg2d = _deg_kernel(col3, ones_chunk, zeros128)
    u0, dinv, dinv2 = _prep(x_pad, deg2d)
    acc1 = _round_kernel(u0, row_p, col_p, zeros128)
    u1 = _scale(acc1, dinv2)
    acc2 = _round_kernel(u1, row_p, col_p, zeros128)
    return _final(acc2, dinv, W.T, b.reshape(1, D))
